# baseline (speedup 1.0000x reference)
import jax
import jax.numpy as jnp
from jax.experimental import pallas as pl
from jax.experimental.pallas import tpu as pltpu

_B = 64
_CIN = 256
_COUT = 256
_SLAB = 128
_CHUNK = 8
_NSTRIPE = _SLAB // _CHUNK
_KSTEPS = _NSTRIPE + 2
_XS = 72
_OS = 40
_QG = 32


def _gconv_kernel(x_ref, w_ref, b_ref, o_ref, xt_ref, ot_ref):
    k = pl.program_id(1)

    @pl.when(k == 0)
    def _xpose_in():
        for b in range(_B):
            t = x_ref[b].T
            xt_ref[0, pl.ds(b, _SLAB, stride=_XS), :] = t[:, :128]
            xt_ref[1, pl.ds(b, _SLAB, stride=_XS), :] = t[:, 128:]

    @pl.when(jnp.logical_and(k >= 1, k <= _NSTRIPE))
    def _stripes():
        s = k - 1
        qbuf = jax.lax.rem(jax.lax.div(s, 4), 2)
        glo = jax.lax.rem(s, 4) * _CHUNK
        base = s * (_CHUNK * _XS)
        for t in range(_CHUNK):
            r = pl.ds(base + t * _XS, _B)
            lhs = jnp.concatenate(
                [xt_ref[0, r, :], xt_ref[1, r, :]], axis=1)
            og = jax.lax.dot_general(
                lhs, w_ref[t],
                dimension_numbers=(((1,), (1,)), ((), ())),
                preferred_element_type=jnp.float32,
            ) + b_ref[t : t + 1, :]
            ro = pl.ds(glo + t, _B, stride=_OS)
            ot_ref[qbuf, 0, ro, :] = og[:, :128]
            ot_ref[qbuf, 1, ro, :] = og[:, 128:]

    for q in range(4):
        @pl.when(k == 4 * q + 5)
        def _flush(q=q):
            for b in range(_B):
                rq = pl.ds(b * _OS, _QG)
                v = jnp.concatenate(
                    [ot_ref[q % 2, 0, rq, :], ot_ref[q % 2, 1, rq, :]],
                    axis=1)
                o_ref[b, :, q * _QG : (q + 1) * _QG] = v.T


def kernel(x, W, b):
    B, Cin, H, Wsp = x.shape
    G = H * Wsp
    Cout = W.shape[1]
    xf = x.reshape(B, Cin, G)
    nslab = G // _SLAB

    def _wmap(j, k):
        return (j * _NSTRIPE + jnp.clip(k - 1, 0, _NSTRIPE - 1), 0, 0)

    out = pl.pallas_call(
        _gconv_kernel,
        grid=(nslab, _KSTEPS),
        in_specs=[
            pl.BlockSpec((B, Cin, _SLAB), lambda j, k: (0, 0, j)),
            pl.BlockSpec((_CHUNK, Cout, Cin), _wmap),
            pl.BlockSpec((_CHUNK, Cout), lambda j, k: (_wmap(j, k)[0], 0)),
        ],
        out_specs=pl.BlockSpec((B, Cout, _SLAB), lambda j, k: (0, 0, j)),
        out_shape=jax.ShapeDtypeStruct((B, Cout, G), jnp.float32),
        scratch_shapes=[
            pltpu.VMEM((2, _SLAB * _XS, 128), jnp.float32),
            pltpu.VMEM((2, 2, (B - 1) * _OS + _QG, 128), jnp.float32),
        ],
        compiler_params=pltpu.CompilerParams(
            dimension_semantics=("parallel", "arbitrary"),
            vmem_limit_bytes=60000 * 1024,
        ),
        name="regroup_conv_fused_scatter",
    )(xf, W, b)
    return out.reshape(B, Cout, H, Wsp)


# speedup vs baseline: 3.3191x; 3.3191x over previous
import jax
import jax.numpy as jnp
from jax.experimental import pallas as pl
from jax.experimental.pallas import tpu as pltpu

_B = 64
_GB = 32
_XS = 72
_OS = 40


def _gconv_kernel(x_ref, w_ref, b_ref, o_ref, xt_ref, ot_ref):
    for b in range(_B):
        v = x_ref[b]
        xt_ref[0, pl.ds(b, _GB, stride=_XS), :] = v[:, :128]
        xt_ref[1, pl.ds(b, _GB, stride=_XS), :] = v[:, 128:]
    for g in range(_GB):
        r = pl.ds(g * _XS, _B)
        lhs = jnp.concatenate(
            [xt_ref[0, r, :], xt_ref[1, r, :]], axis=1)
        og = jax.lax.dot_general(
            lhs, w_ref[g],
            dimension_numbers=(((1,), (1,)), ((), ())),
            preferred_element_type=jnp.float32,
        ) + b_ref[g : g + 1, :]
        ro = pl.ds(g, _B, stride=_OS)
        ot_ref[0, ro, :] = og[:, :128]
        ot_ref[1, ro, :] = og[:, 128:]
    for b in range(_B):
        rq = pl.ds(b * _OS, _GB)
        o_ref[b] = jnp.concatenate(
            [ot_ref[0, rq, :], ot_ref[1, rq, :]], axis=1)


def kernel(x, W, b):
    B, Cin, H, Wsp = x.shape
    G = H * Wsp
    Cout = W.shape[1]
    xp = jnp.transpose(x, (0, 2, 3, 1)).reshape(B, G, Cin)
    out = pl.pallas_call(
        _gconv_kernel,
        grid=(G // _GB,),
        in_specs=[
            pl.BlockSpec((B, _GB, Cin), lambda j: (0, j, 0)),
            pl.BlockSpec((_GB, Cout, Cin), lambda j: (j, 0, 0)),
            pl.BlockSpec((_GB, Cout), lambda j: (j, 0)),
        ],
        out_specs=pl.BlockSpec((B, _GB, Cout), lambda j: (0, j, 0)),
        out_shape=jax.ShapeDtypeStruct((B, G, Cout), jnp.float32),
        scratch_shapes=[
            pltpu.VMEM((2, (_GB - 1) * _XS + _B, 128), jnp.float32),
            pltpu.VMEM((2, (_B - 1) * _OS + _GB, 128), jnp.float32),
        ],
        compiler_params=pltpu.CompilerParams(
            dimension_semantics=("parallel",),
            vmem_limit_bytes=60000 * 1024,
        ),
        name="regroup_conv_cl",
    )(xp, W, b)
    return jnp.transpose(out.reshape(B, H, Wsp, Cout), (0, 3, 1, 2))
